# baseline (speedup 1.0000x reference)
import functools

import jax
import jax.numpy as jnp
from jax import lax
from jax.experimental import pallas as pl
from jax.experimental.pallas import tpu as pltpu
import jax.experimental.pallas.tpu_sc as plsc

_B, _L = 4096, 200
_N = _B * _L
_D = 64
_M = 128
_NP = 512
_NW = 32
_RPW = _N // _NW
_C = 128
_NCH = _RPW // _C
_IR = _RPW // 128
_LP = 208


def _fused(x2, table, pw, gamma, beta):
    mesh = plsc.VectorSubcoreMesh(core_axis_name="c", subcore_axis_name="s")

    @functools.partial(
        pl.kernel,
        out_type=jax.ShapeDtypeStruct((_N, _M), jnp.float32),
        mesh=mesh,
        scratch_types=[
            pltpu.VMEM((_IR, 128), jnp.int32),
            pltpu.VMEM((_D, _NP), jnp.float32),
            pltpu.VMEM((_M,), jnp.float32),
            pltpu.VMEM((_M,), jnp.float32),
            pltpu.VMEM((_LP, 16), jnp.float32),
            pltpu.VMEM((_C, _D), jnp.float32),
            pltpu.VMEM((_C, _D), jnp.float32),
            pltpu.VMEM((_C, _M), jnp.float32),
            pltpu.VMEM((_C, _M), jnp.float32),
            pltpu.SemaphoreType.DMA,
            pltpu.SemaphoreType.DMA,
            pltpu.SemaphoreType.DMA,
            pltpu.SemaphoreType.DMA,
        ],
        compiler_params=pltpu.CompilerParams(
            use_tc_tiling_on_sc=False, needs_layout_passes=False),
    )
    def k(x_hbm, tab_hbm, pw_hbm, g_hbm, b_hbm, out_hbm,
          idx_all, PW_v, g_v, b_v, PST_v, rows0, rows1, outb0, outb1,
          gsem0, gsem1, wsem0, wsem1):
        wid = lax.axis_index("s") * 2 + lax.axis_index("c")
        base = pl.multiple_of(wid * _RPW, _C)

        pltpu.sync_copy(
            x_hbm.at[pl.ds(pl.multiple_of(wid * _IR, 8), _IR)], idx_all)
        pltpu.sync_copy(pw_hbm, PW_v)
        pltpu.sync_copy(g_hbm, g_v)
        pltpu.sync_copy(b_hbm, b_v)

        iota16 = lax.iota(jnp.int32, 16)
        c0 = jnp.zeros((16,), jnp.int32)
        c1 = jnp.ones((16,), jnp.int32)
        cmagic = jnp.full((16,), 0x5F3759DF, jnp.int32)

        def pstat_group(lg, carry):
            lvec = lg * 16 + iota16

            f0 = jnp.zeros((16,), jnp.float32)

            def acc(d, sqd):
                s, q, dv = sqd
                t = plsc.load_gather(PW_v, [dv, lvec])
                return (s + t, q + t * t, dv + 1)

            s, q, _ = lax.fori_loop(0, _D, acc, (f0, f0, c0))
            plsc.store_scatter(PST_v, [lvec, c0], s)
            plsc.store_scatter(PST_v, [lvec, c1], q)
            return carry

        lax.fori_loop(0, _LP // 16, pstat_group, 0)

        gv = [g_v[pl.ds(t * 16, 16)] for t in range(8)]
        bv = [b_v[pl.ds(t * 16, 16)] for t in range(8)]
        ct16 = [jnp.full((16,), t * 16, jnp.int32) + iota16 for t in range(4)]

        rows = (rows0, rows1)
        outs = (outb0, outb1)
        gsems = (gsem0, gsem1)
        wsems = (wsem0, wsem1)

        def start_gather(i, c):
            pltpu.async_copy(tab_hbm.at[idx_all.at[i]], rows[c], gsems[c])

        def wait_gather(c):
            pltpu.make_async_copy(
                tab_hbm.at[idx_all.at[0]], rows[c], gsems[c]).wait()

        def wait_write(c):
            pltpu.make_async_copy(
                outs[c], out_hbm.at[pl.ds(0, _C)], wsems[c]).wait()

        def compute(c, i):
            r_ref = rows[c]
            o_ref = outs[c]
            l0 = lax.rem(i * _C, _L)
            lvec0 = l0 + iota16
            lvec0 = jnp.where(lvec0 >= _L, lvec0 - _L, lvec0)

            def group(g, lvec):
                rvec = g * 16 + iota16
                s = [None] * 4
                q = [None] * 4
                dvs = [c0 + u for u in range(4)]
                for u in range(4):
                    t = plsc.load_gather(r_ref, [rvec, dvs[u]])
                    s[u] = t
                    q[u] = t * t
                    dvs[u] = dvs[u] + 4
                for d in range(4, _D, 4):
                    for u in range(4):
                        t = plsc.load_gather(r_ref, [rvec, dvs[u]])
                        s[u] = s[u] + t
                        q[u] = q[u] + t * t
                        dvs[u] = dvs[u] + 4
                S = (s[0] + s[1]) + (s[2] + s[3])
                Q = (q[0] + q[1]) + (q[2] + q[3])
                Sp = plsc.load_gather(PST_v, [lvec, c0])
                Qp = plsc.load_gather(PST_v, [lvec, c1])
                mu = (S + Sp) * (1.0 / _M)
                var = (Q + Qp) * (1.0 / _M) - mu * mu + 1e-5
                seed = cmagic - lax.shift_right_logical(
                    plsc.bitcast(var, jnp.int32), 1)
                y = plsc.bitcast(seed, jnp.float32)
                hx = 0.5 * var
                y = y * (1.5 - hx * y * y)
                y = y * (1.5 - hx * y * y)
                t1 = mu * y

                @plsc.parallel_loop(0, 16, 1, unroll=4,
                                    carry=jnp.zeros((16,), jnp.int32))
                def rowp(j, jv):
                    jj = g * 16 + j
                    e = [r_ref[jj, pl.ds(t * 16, 16)] for t in range(4)]
                    ybc = y.at[jv].get(mode="promise_in_bounds")
                    tbc = t1.at[jv].get(mode="promise_in_bounds")
                    lbc = lvec.at[jv].get(mode="promise_in_bounds")
                    p = [plsc.load_gather(PW_v, [ct16[t], lbc])
                         for t in range(4)]
                    for t in range(4):
                        o_ref[jj, pl.ds(t * 16, 16)] = (
                            (e[t] * ybc - tbc) * gv[t] + bv[t])
                    for t in range(4):
                        o_ref[jj, pl.ds(_D + t * 16, 16)] = (
                            (p[t] * ybc - tbc) * gv[4 + t] + bv[4 + t])
                    return jv + 1

                lv2 = lvec + 16
                return jnp.where(lv2 >= _L, lv2 - _L, lv2)

            lax.fori_loop(0, _C // 16, group, lvec0)

        start_gather(0, 0)
        start_gather(1, 1)

        def step(ii, carry):
            for sub in range(2):
                i = ii * 2 + sub
                c = sub
                wait_gather(c)

                @pl.when(i >= 2)
                def _():
                    wait_write(c)

                compute(c, i)
                b = pl.multiple_of(base + i * _C, _C)
                pltpu.async_copy(outs[c], out_hbm.at[pl.ds(b, _C)], wsems[c])

                @pl.when(i + 2 < _NCH)
                def _():
                    start_gather(i + 2, c)

            return carry

        lax.fori_loop(0, _NCH // 2, step, 0)
        wait_write(0)
        wait_write(1)

    return k(x2, table, pw, gamma, beta)


def kernel(x, table, pos_weight, ln_gamma, ln_beta):
    x2 = x.astype(jnp.int32).reshape(_N // 128, 128)
    h = _fused(x2, table, pos_weight, ln_gamma, ln_beta)
    return h.reshape(_B, _L, _M)


# speedup vs baseline: 3.6002x; 3.6002x over previous
import functools

import jax
import jax.numpy as jnp
from jax import lax
from jax.experimental import pallas as pl
from jax.experimental.pallas import tpu as pltpu
import jax.experimental.pallas.tpu_sc as plsc

_B, _L = 4096, 200
_N = _B * _L
_D = 64
_DA = 80
_M = 128
_V = 100000
_NW = 32
_RPW = _N // _NW
_C = 128
_NCH = _RPW // _C
_IR = _RPW // 128


def _prep_body(t_ref, p_ref, aug_ref, pst_ref):
    t = t_ref[...]
    s = jnp.sum(t, axis=-1, keepdims=True)
    q = jnp.sum(t * t, axis=-1, keepdims=True)
    pad = jnp.zeros((t.shape[0], _DA - _D - 2), jnp.float32)
    aug_ref[...] = jnp.concatenate([t, s, q, pad], axis=-1)
    p = p_ref[...]
    sp = jnp.sum(p, axis=-1, keepdims=True)
    qp = jnp.sum(p * p, axis=-1, keepdims=True)
    ppad = jnp.zeros((p.shape[0], 14), jnp.float32)
    pst_ref[...] = jnp.concatenate([sp, qp, ppad], axis=-1)


def _tc_prep(table, posT):
    R = 2000
    return pl.pallas_call(
        _prep_body,
        grid=(_V // R,),
        in_specs=[pl.BlockSpec((R, _D), lambda i: (i, 0)),
                  pl.BlockSpec((_L, _D), lambda i: (0, 0))],
        out_specs=[pl.BlockSpec((R, _DA), lambda i: (i, 0)),
                   pl.BlockSpec((_L, 16), lambda i: (0, 0))],
        out_shape=[jax.ShapeDtypeStruct((_V, _DA), jnp.float32),
                   jax.ShapeDtypeStruct((_L, 16), jnp.float32)],
    )(table, posT)


def _fused(x2, aug_tab, posT, pst, gamma, beta):
    mesh = plsc.VectorSubcoreMesh(core_axis_name="c", subcore_axis_name="s")

    @functools.partial(
        pl.kernel,
        out_type=jax.ShapeDtypeStruct((_N, _M), jnp.float32),
        mesh=mesh,
        scratch_types=[
            pltpu.VMEM((_IR, 128), jnp.int32),
            pltpu.VMEM((_L, _D), jnp.float32),
            pltpu.VMEM((_M,), jnp.float32),
            pltpu.VMEM((_M,), jnp.float32),
            pltpu.VMEM((_L, 16), jnp.float32),
            pltpu.VMEM((_C, _DA), jnp.float32),
            pltpu.VMEM((_C, _DA), jnp.float32),
            pltpu.VMEM((_C, _M), jnp.float32),
            pltpu.VMEM((_C, _M), jnp.float32),
            pltpu.SemaphoreType.DMA,
            pltpu.SemaphoreType.DMA,
            pltpu.SemaphoreType.DMA,
            pltpu.SemaphoreType.DMA,
        ],
        compiler_params=pltpu.CompilerParams(
            use_tc_tiling_on_sc=False, needs_layout_passes=False),
    )
    def k(x_hbm, tab_hbm, pos_hbm, pst_hbm, g_hbm, b_hbm, out_hbm,
          idx_all, P_v, g_v, b_v, PST_v, rows0, rows1, outb0, outb1,
          gsem0, gsem1, wsem0, wsem1):
        wid = lax.axis_index("s") * 2 + lax.axis_index("c")
        base = pl.multiple_of(wid * _RPW, _C)

        pltpu.sync_copy(
            x_hbm.at[pl.ds(pl.multiple_of(wid * _IR, 8), _IR)], idx_all)
        pltpu.sync_copy(pos_hbm, P_v)
        pltpu.sync_copy(pst_hbm, PST_v)
        pltpu.sync_copy(g_hbm, g_v)
        pltpu.sync_copy(b_hbm, b_v)

        iota16 = lax.iota(jnp.int32, 16)
        c64 = jnp.full((16,), _D, jnp.int32)
        c65 = jnp.full((16,), _D + 1, jnp.int32)
        c0 = jnp.zeros((16,), jnp.int32)
        c1 = jnp.ones((16,), jnp.int32)
        cmagic = jnp.full((16,), 0x5F3759DF, jnp.int32)

        gv = [g_v[pl.ds(t * 16, 16)] for t in range(8)]
        bv = [b_v[pl.ds(t * 16, 16)] for t in range(8)]

        rows = (rows0, rows1)
        outs = (outb0, outb1)
        gsems = (gsem0, gsem1)
        wsems = (wsem0, wsem1)

        def start_gather(i, c):
            pltpu.async_copy(tab_hbm.at[idx_all.at[i]], rows[c], gsems[c])

        def wait_gather(c):
            pltpu.make_async_copy(
                tab_hbm.at[idx_all.at[0]], rows[c], gsems[c]).wait()

        def wait_write(c):
            pltpu.make_async_copy(
                outs[c], out_hbm.at[pl.ds(0, _C)], wsems[c]).wait()

        def compute(c, i):
            r_ref = rows[c]
            o_ref = outs[c]
            l0 = lax.rem(i * _C, _L)
            lvec0 = l0 + iota16
            lvec0 = jnp.where(lvec0 >= _L, lvec0 - _L, lvec0)

            def group(g, lvec):
                rvec = g * 16 + iota16
                S = plsc.load_gather(r_ref, [rvec, c64])
                Q = plsc.load_gather(r_ref, [rvec, c65])
                Sp = plsc.load_gather(PST_v, [lvec, c0])
                Qp = plsc.load_gather(PST_v, [lvec, c1])
                mu = (S + Sp) * (1.0 / _M)
                var = (Q + Qp) * (1.0 / _M) - mu * mu + 1e-5
                seed = cmagic - lax.shift_right_logical(
                    plsc.bitcast(var, jnp.int32), 1)
                y = plsc.bitcast(seed, jnp.float32)
                hx = 0.5 * var
                y = y * (1.5 - hx * y * y)
                y = y * (1.5 - hx * y * y)
                t1 = mu * y

                @plsc.parallel_loop(0, 16, 1, unroll=4,
                                    carry=jnp.zeros((16,), jnp.int32))
                def rowp(j, jv):
                    jj = g * 16 + j
                    lj = l0 + jj
                    l = jnp.where(lj >= _L, lj - _L, lj)
                    e = [r_ref[jj, pl.ds(t * 16, 16)] for t in range(4)]
                    p = [P_v[l, pl.ds(t * 16, 16)] for t in range(4)]
                    ybc = y.at[jv].get(mode="promise_in_bounds")
                    tbc = t1.at[jv].get(mode="promise_in_bounds")
                    for t in range(4):
                        o_ref[jj, pl.ds(t * 16, 16)] = (
                            (e[t] * ybc - tbc) * gv[t] + bv[t])
                    for t in range(4):
                        o_ref[jj, pl.ds(_D + t * 16, 16)] = (
                            (p[t] * ybc - tbc) * gv[4 + t] + bv[4 + t])
                    return jv + 1

                lv2 = lvec + 16
                return jnp.where(lv2 >= _L, lv2 - _L, lv2)

            lax.fori_loop(0, _C // 16, group, lvec0)

        start_gather(0, 0)
        start_gather(1, 1)

        def step(ii, carry):
            for sub in range(2):
                i = ii * 2 + sub
                c = sub
                wait_gather(c)

                @pl.when(i >= 2)
                def _():
                    wait_write(c)

                compute(c, i)
                b = pl.multiple_of(base + i * _C, _C)
                pltpu.async_copy(outs[c], out_hbm.at[pl.ds(b, _C)], wsems[c])

                @pl.when(i + 2 < _NCH)
                def _():
                    start_gather(i + 2, c)

            return carry

        lax.fori_loop(0, _NCH // 2, step, 0)
        wait_write(0)
        wait_write(1)

    return k(x2, aug_tab, posT, pst, gamma, beta)


def kernel(x, table, pos_weight, ln_gamma, ln_beta):
    x2 = x.astype(jnp.int32).reshape(_N // 128, 128)
    posT = pos_weight[:, :_L].T
    aug, pst = _tc_prep(table, posT)
    h = _fused(x2, aug, posT, pst, ln_gamma, ln_beta)
    return h.reshape(_B, _L, _M)


# speedup vs baseline: 3.9206x; 1.0890x over previous
import functools

import jax
import jax.numpy as jnp
from jax import lax
from jax.experimental import pallas as pl
from jax.experimental.pallas import tpu as pltpu
import jax.experimental.pallas.tpu_sc as plsc

_B, _L = 4096, 200
_N = _B * _L
_D = 64
_DA = 80
_M = 128
_V = 100000
_NP = 512
_NW = 32
_SPW = _B // _NW
_C = _L
_LP = 208


def _prep_body(t_ref, pw_ref, aug_ref, posT_ref, pst_ref):
    t = t_ref[...]
    s = jnp.sum(t, axis=-1, keepdims=True)
    q = jnp.sum(t * t, axis=-1, keepdims=True)
    pad = jnp.zeros((t.shape[0], _DA - _D - 2), jnp.float32)
    aug_ref[...] = jnp.concatenate([t, s, q, pad], axis=-1)

    @pl.when(pl.program_id(0) == 0)
    def _():
        p = pw_ref[...].T[:_L]
        posT_ref[...] = p
        sp = jnp.sum(p, axis=-1, keepdims=True)
        qp = jnp.sum(p * p, axis=-1, keepdims=True)
        ppad = jnp.zeros((_L, 14), jnp.float32)
        pst_ref[...] = jnp.concatenate([sp, qp, ppad], axis=-1)


def _tc_prep(table, pos_weight):
    R = 5000
    return pl.pallas_call(
        _prep_body,
        grid=(_V // R,),
        in_specs=[pl.BlockSpec((R, _D), lambda i: (i, 0)),
                  pl.BlockSpec((_D, _NP), lambda i: (0, 0))],
        out_specs=[pl.BlockSpec((R, _DA), lambda i: (i, 0)),
                   pl.BlockSpec((_L, _D), lambda i: (0, 0)),
                   pl.BlockSpec((_L, 16), lambda i: (0, 0))],
        out_shape=[jax.ShapeDtypeStruct((_V, _DA), jnp.float32),
                   jax.ShapeDtypeStruct((_L, _D), jnp.float32),
                   jax.ShapeDtypeStruct((_L, 16), jnp.float32)],
    )(table, pos_weight)


def _fused(x, aug_tab, posT, pst, gamma, beta):
    mesh = plsc.VectorSubcoreMesh(core_axis_name="c", subcore_axis_name="s")

    @functools.partial(
        pl.kernel,
        out_type=jax.ShapeDtypeStruct((_B, _L, _M), jnp.float32),
        mesh=mesh,
        scratch_types=[
            pltpu.VMEM((_SPW, _L), jnp.int32),
            pltpu.VMEM((_L, _D), jnp.float32),
            pltpu.VMEM((_M,), jnp.float32),
            pltpu.VMEM((_M,), jnp.float32),
            pltpu.VMEM((_LP, 16), jnp.float32),
            pltpu.VMEM((_LP, _DA), jnp.float32),
            pltpu.VMEM((_LP, _DA), jnp.float32),
            pltpu.VMEM((_C, _M), jnp.float32),
            pltpu.VMEM((_C, _M), jnp.float32),
            pltpu.SemaphoreType.DMA,
            pltpu.SemaphoreType.DMA,
            pltpu.SemaphoreType.DMA,
            pltpu.SemaphoreType.DMA,
        ],
        compiler_params=pltpu.CompilerParams(
            use_tc_tiling_on_sc=False, needs_layout_passes=False),
    )
    def k(x_hbm, tab_hbm, pos_hbm, pst_hbm, g_hbm, b_hbm, out_hbm,
          idx_all, P_v, g_v, b_v, PST_v, rows0, rows1, outb0, outb1,
          gsem0, gsem1, wsem0, wsem1):
        wid = lax.axis_index("s") * 2 + lax.axis_index("c")
        sbase = pl.multiple_of(wid * _SPW, 8)

        pltpu.sync_copy(x_hbm.at[pl.ds(sbase, _SPW)], idx_all)
        pltpu.sync_copy(pos_hbm, P_v)
        pltpu.sync_copy(pst_hbm, PST_v.at[pl.ds(0, _L)])
        pltpu.sync_copy(g_hbm, g_v)
        pltpu.sync_copy(b_hbm, b_v)

        iota16 = lax.iota(jnp.int32, 16)
        c64 = jnp.full((16,), _D, jnp.int32)
        c65 = jnp.full((16,), _D + 1, jnp.int32)
        c0 = jnp.zeros((16,), jnp.int32)
        c1 = jnp.ones((16,), jnp.int32)
        cmagic = jnp.full((16,), 0x5F3759DF, jnp.int32)

        gv = [g_v[pl.ds(t * 16, 16)] for t in range(8)]
        bv = [b_v[pl.ds(t * 16, 16)] for t in range(8)]

        rows = (rows0, rows1)
        outs = (outb0, outb1)
        gsems = (gsem0, gsem1)
        wsems = (wsem0, wsem1)

        def start_gather(i, c):
            pltpu.async_copy(tab_hbm.at[idx_all.at[i, pl.ds(0, 128)]],
                             rows[c].at[pl.ds(0, 128)], gsems[c])
            pltpu.async_copy(tab_hbm.at[idx_all.at[i, pl.ds(128, 72)]],
                             rows[c].at[pl.ds(128, 72)], gsems[c])

        def wait_gather(c):
            pltpu.make_async_copy(
                tab_hbm.at[idx_all.at[0, pl.ds(0, 128)]],
                rows[c].at[pl.ds(0, 128)], gsems[c]).wait()
            pltpu.make_async_copy(
                tab_hbm.at[idx_all.at[0, pl.ds(128, 72)]],
                rows[c].at[pl.ds(128, 72)], gsems[c]).wait()

        def wait_write(c):
            pltpu.make_async_copy(outs[c], out_hbm.at[0], wsems[c]).wait()

        def compute(c):
            r_ref = rows[c]
            o_ref = outs[c]

            def do_group(r0, nrow):
                lvec = r0 + iota16
                S = plsc.load_gather(r_ref, [lvec, c64])
                Q = plsc.load_gather(r_ref, [lvec, c65])
                Sp = plsc.load_gather(PST_v, [lvec, c0])
                Qp = plsc.load_gather(PST_v, [lvec, c1])
                mu = (S + Sp) * (1.0 / _M)
                var = (Q + Qp) * (1.0 / _M) - mu * mu + 1e-5
                seed = cmagic - lax.shift_right_logical(
                    plsc.bitcast(var, jnp.int32), 1)
                y = plsc.bitcast(seed, jnp.float32)
                hx = 0.5 * var
                y = y * (1.5 - hx * y * y)
                y = y * (1.5 - hx * y * y)
                t1 = mu * y

                @plsc.parallel_loop(0, nrow, 1, unroll=4,
                                    carry=jnp.zeros((16,), jnp.int32))
                def rowp(j, jv):
                    jj = r0 + j
                    e = [r_ref[jj, pl.ds(t * 16, 16)] for t in range(4)]
                    p = [P_v[jj, pl.ds(t * 16, 16)] for t in range(4)]
                    ybc = y.at[jv].get(mode="promise_in_bounds")
                    tbc = t1.at[jv].get(mode="promise_in_bounds")
                    for t in range(4):
                        o_ref[jj, pl.ds(t * 16, 16)] = (
                            (e[t] * ybc - tbc) * gv[t] + bv[t])
                    for t in range(4):
                        o_ref[jj, pl.ds(_D + t * 16, 16)] = (
                            (p[t] * ybc - tbc) * gv[4 + t] + bv[4 + t])
                    return jv + 1

            def group(g, carry):
                do_group(g * 16, 16)
                return carry

            lax.fori_loop(0, _L // 16, group, 0)
            do_group(192, _L - 16 * (_L // 16))

        start_gather(0, 0)
        start_gather(1, 1)

        def step(ii, carry):
            for sub in range(2):
                i = ii * 2 + sub
                c = sub
                wait_gather(c)

                @pl.when(i >= 2)
                def _():
                    wait_write(c)

                compute(c)
                pltpu.async_copy(outs[c], out_hbm.at[sbase + i], wsems[c])

                @pl.when(i + 2 < _SPW)
                def _():
                    start_gather(i + 2, c)

            return carry

        lax.fori_loop(0, _SPW // 2, step, 0)
        wait_write(0)
        wait_write(1)

    return k(x, aug_tab, posT, pst, gamma, beta)


def kernel(x, table, pos_weight, ln_gamma, ln_beta):
    aug, posT, pst = _tc_prep(table, pos_weight)
    return _fused(x.astype(jnp.int32), aug, posT, pst, ln_gamma, ln_beta)


# speedup vs baseline: 4.0606x; 1.0357x over previous
import functools

import jax
import jax.numpy as jnp
from jax import lax
from jax.experimental import pallas as pl
from jax.experimental.pallas import tpu as pltpu
import jax.experimental.pallas.tpu_sc as plsc

_B, _L = 4096, 200
_N = _B * _L
_D = 64
_DA = 128
_M = 128
_V = 100000
_NP = 512
_NW = 32
_SPW = _B // _NW
_LP = 208


def _prep_body(t_ref, pw_ref, aug_ref, posT_ref, pst_ref):
    t = t_ref[...]
    s = jnp.sum(t, axis=-1, keepdims=True)
    q = jnp.sum(t * t, axis=-1, keepdims=True)
    pad = jnp.zeros((t.shape[0], _DA - _D - 2), jnp.float32)
    aug_ref[...] = jnp.concatenate([t, s, q, pad], axis=-1)

    @pl.when(pl.program_id(0) == 0)
    def _():
        p = pw_ref[...].T[:_L]
        posT_ref[...] = p
        sp = jnp.sum(p, axis=-1, keepdims=True)
        qp = jnp.sum(p * p, axis=-1, keepdims=True)
        ppad = jnp.zeros((_L, 14), jnp.float32)
        pst_ref[...] = jnp.concatenate([sp, qp, ppad], axis=-1)


def _tc_prep(table, pos_weight):
    R = 5000
    return pl.pallas_call(
        _prep_body,
        grid=(_V // R,),
        in_specs=[pl.BlockSpec((R, _D), lambda i: (i, 0)),
                  pl.BlockSpec((_D, _NP), lambda i: (0, 0))],
        out_specs=[pl.BlockSpec((R, _DA), lambda i: (i, 0)),
                   pl.BlockSpec((_L, _D), lambda i: (0, 0)),
                   pl.BlockSpec((_L, 16), lambda i: (0, 0))],
        out_shape=[jax.ShapeDtypeStruct((_V, _DA), jnp.float32),
                   jax.ShapeDtypeStruct((_L, _D), jnp.float32),
                   jax.ShapeDtypeStruct((_L, 16), jnp.float32)],
    )(table, pos_weight)


def _fused(x, aug_tab, posT, pst, gamma, beta):
    mesh = plsc.VectorSubcoreMesh(core_axis_name="c", subcore_axis_name="s")

    @functools.partial(
        pl.kernel,
        out_type=jax.ShapeDtypeStruct((_B, _L, _M), jnp.float32),
        mesh=mesh,
        scratch_types=[
            pltpu.VMEM((_SPW, _L), jnp.int32),
            pltpu.VMEM((_L, _D), jnp.float32),
            pltpu.VMEM((_M,), jnp.float32),
            pltpu.VMEM((_M,), jnp.float32),
            pltpu.VMEM((_LP, 16), jnp.float32),
            pltpu.VMEM((_LP, _DA), jnp.float32),
            pltpu.VMEM((_LP, _DA), jnp.float32),
            pltpu.VMEM((_LP, _DA), jnp.float32),
            pltpu.SemaphoreType.DMA,
            pltpu.SemaphoreType.DMA,
            pltpu.SemaphoreType.DMA,
            pltpu.SemaphoreType.DMA,
            pltpu.SemaphoreType.DMA,
            pltpu.SemaphoreType.DMA,
        ],
        compiler_params=pltpu.CompilerParams(
            use_tc_tiling_on_sc=False, needs_layout_passes=False),
    )
    def k(x_hbm, tab_hbm, pos_hbm, pst_hbm, g_hbm, b_hbm, out_hbm,
          idx_all, P_v, g_v, b_v, PST_v, buf0, buf1, buf2,
          gsem0, gsem1, gsem2, wsem0, wsem1, wsem2):
        wid = lax.axis_index("s") * 2 + lax.axis_index("c")
        sbase = pl.multiple_of(wid * _SPW, 8)

        pltpu.sync_copy(x_hbm.at[pl.ds(sbase, _SPW)], idx_all)
        pltpu.sync_copy(pos_hbm, P_v)
        pltpu.sync_copy(pst_hbm, PST_v.at[pl.ds(0, _L)])
        pltpu.sync_copy(g_hbm, g_v)
        pltpu.sync_copy(b_hbm, b_v)

        iota16 = lax.iota(jnp.int32, 16)
        c64 = jnp.full((16,), _D, jnp.int32)
        c65 = jnp.full((16,), _D + 1, jnp.int32)
        c0 = jnp.zeros((16,), jnp.int32)
        c1 = jnp.ones((16,), jnp.int32)
        cmagic = jnp.full((16,), 0x5F3759DF, jnp.int32)

        gv = [g_v[pl.ds(t * 16, 16)] for t in range(8)]
        bv = [b_v[pl.ds(t * 16, 16)] for t in range(8)]

        bufs = (buf0, buf1, buf2)
        gsems = (gsem0, gsem1, gsem2)
        wsems = (wsem0, wsem1, wsem2)

        def start_gather(i, c):
            pltpu.async_copy(tab_hbm.at[idx_all.at[i, pl.ds(0, 128)]],
                             bufs[c].at[pl.ds(0, 128)], gsems[c])
            pltpu.async_copy(tab_hbm.at[idx_all.at[i, pl.ds(128, 72)]],
                             bufs[c].at[pl.ds(128, 72)], gsems[c])

        def wait_gather(c):
            pltpu.make_async_copy(
                tab_hbm.at[idx_all.at[0, pl.ds(0, 128)]],
                bufs[c].at[pl.ds(0, 128)], gsems[c]).wait()
            pltpu.make_async_copy(
                tab_hbm.at[idx_all.at[0, pl.ds(128, 72)]],
                bufs[c].at[pl.ds(128, 72)], gsems[c]).wait()

        def start_write(i, c):
            pltpu.async_copy(bufs[c].at[pl.ds(0, _L)],
                             out_hbm.at[sbase + i], wsems[c])

        def wait_write(c):
            pltpu.make_async_copy(bufs[c].at[pl.ds(0, _L)],
                                  out_hbm.at[0], wsems[c]).wait()

        def compute(c):
            r_ref = bufs[c]

            def do_group(r0, nrow):
                lvec = r0 + iota16
                S = plsc.load_gather(r_ref, [lvec, c64])
                Q = plsc.load_gather(r_ref, [lvec, c65])
                Sp = plsc.load_gather(PST_v, [lvec, c0])
                Qp = plsc.load_gather(PST_v, [lvec, c1])
                mu = (S + Sp) * (1.0 / _M)
                var = (Q + Qp) * (1.0 / _M) - mu * mu + 1e-5
                seed = cmagic - lax.shift_right_logical(
                    plsc.bitcast(var, jnp.int32), 1)
                y = plsc.bitcast(seed, jnp.float32)
                hx = 0.5 * var
                y = y * (1.5 - hx * y * y)
                y = y * (1.5 - hx * y * y)
                t1 = mu * y

                @plsc.parallel_loop(0, nrow, 1, unroll=4,
                                    carry=jnp.zeros((16,), jnp.int32))
                def rowp(j, jv):
                    jj = r0 + j
                    e = [r_ref[jj, pl.ds(t * 16, 16)] for t in range(4)]
                    p = [P_v[jj, pl.ds(t * 16, 16)] for t in range(4)]
                    ybc = y.at[jv].get(mode="promise_in_bounds")
                    tbc = t1.at[jv].get(mode="promise_in_bounds")
                    for t in range(4):
                        r_ref[jj, pl.ds(t * 16, 16)] = (
                            (e[t] * ybc - tbc) * gv[t] + bv[t])
                    for t in range(4):
                        r_ref[jj, pl.ds(_D + t * 16, 16)] = (
                            (p[t] * ybc - tbc) * gv[4 + t] + bv[4 + t])
                    return jv + 1

            def group(g, carry):
                do_group(g * 16, 16)
                return carry

            lax.fori_loop(0, _L // 16, group, 0)
            do_group(192, _L - 16 * (_L // 16))

        start_gather(0, 0)
        start_gather(1, 1)
        start_gather(2, 2)

        def step(ii, carry):
            for sub in range(3):
                i = ii * 3 + sub
                c = sub
                b = (sub + 2) % 3

                @pl.when(i < _SPW)
                def _():
                    wait_gather(c)
                    compute(c)
                    start_write(i, c)

                    @pl.when(jnp.logical_and(i >= 1, i + 2 < _SPW))
                    def _():
                        wait_write(b)
                        start_gather(i + 2, b)

            return carry

        lax.fori_loop(0, (_SPW + 2) // 3, step, 0)
        wait_write(0)
        wait_write(1)
        wait_write(2)

    return k(x, aug_tab, posT, pst, gamma, beta)


def kernel(x, table, pos_weight, ln_gamma, ln_beta):
    aug, posT, pst = _tc_prep(table, pos_weight)
    return _fused(x.astype(jnp.int32), aug, posT, pst, ln_gamma, ln_beta)


# speedup vs baseline: 4.1036x; 1.0106x over previous
import functools

import jax
import jax.numpy as jnp
from jax import lax
from jax.experimental import pallas as pl
from jax.experimental.pallas import tpu as pltpu
import jax.experimental.pallas.tpu_sc as plsc

_B, _L = 4096, 200
_N = _B * _L
_D = 64
_DA = 128
_M = 128
_V = 100000
_NP = 512
_NW = 32
_SPW = _B // _NW
_LP = 208


def _prep_body(t_ref, pw_ref, aug_ref, posT_ref, pst_ref):
    t = t_ref[...]
    s = jnp.sum(t, axis=-1, keepdims=True)
    q = jnp.sum(t * t, axis=-1, keepdims=True)
    pad = jnp.zeros((t.shape[0], _DA - _D - 2), jnp.float32)
    aug_ref[...] = jnp.concatenate([t, s, q, pad], axis=-1)

    @pl.when(pl.program_id(0) == 0)
    def _():
        p = pw_ref[...].T[:_L]
        posT_ref[...] = p
        sp = jnp.sum(p, axis=-1, keepdims=True)
        qp = jnp.sum(p * p, axis=-1, keepdims=True)
        ppad = jnp.zeros((_L, 14), jnp.float32)
        pst_ref[...] = jnp.concatenate([sp, qp, ppad], axis=-1)


def _tc_prep(table, pos_weight):
    R = 10000
    return pl.pallas_call(
        _prep_body,
        grid=(_V // R,),
        in_specs=[pl.BlockSpec((R, _D), lambda i: (i, 0)),
                  pl.BlockSpec((_D, _NP), lambda i: (0, 0))],
        out_specs=[pl.BlockSpec((R, _DA), lambda i: (i, 0)),
                   pl.BlockSpec((_L, _D), lambda i: (0, 0)),
                   pl.BlockSpec((_L, 16), lambda i: (0, 0))],
        out_shape=[jax.ShapeDtypeStruct((_V, _DA), jnp.float32),
                   jax.ShapeDtypeStruct((_L, _D), jnp.float32),
                   jax.ShapeDtypeStruct((_L, 16), jnp.float32)],
    )(table, pos_weight)


def _fused(x, aug_tab, posT, pst, gamma, beta):
    mesh = plsc.VectorSubcoreMesh(core_axis_name="c", subcore_axis_name="s")

    @functools.partial(
        pl.kernel,
        out_type=jax.ShapeDtypeStruct((_B, _L, _M), jnp.float32),
        mesh=mesh,
        scratch_types=[
            pltpu.VMEM((_SPW, _L), jnp.int32),
            pltpu.VMEM((_L, _D), jnp.float32),
            pltpu.VMEM((_M,), jnp.float32),
            pltpu.VMEM((_M,), jnp.float32),
            pltpu.VMEM((_LP, 16), jnp.float32),
            pltpu.VMEM((_LP, _DA), jnp.float32),
            pltpu.VMEM((_LP, _DA), jnp.float32),
            pltpu.VMEM((_LP, _DA), jnp.float32),
            pltpu.SemaphoreType.DMA,
            pltpu.SemaphoreType.DMA,
            pltpu.SemaphoreType.DMA,
            pltpu.SemaphoreType.DMA,
            pltpu.SemaphoreType.DMA,
            pltpu.SemaphoreType.DMA,
        ],
        compiler_params=pltpu.CompilerParams(
            use_tc_tiling_on_sc=False, needs_layout_passes=False),
    )
    def k(x_hbm, tab_hbm, pos_hbm, pst_hbm, g_hbm, b_hbm, out_hbm,
          idx_all, P_v, g_v, b_v, PST_v, buf0, buf1, buf2,
          gsem0, gsem1, gsem2, wsem0, wsem1, wsem2):
        wid = lax.axis_index("s") * 2 + lax.axis_index("c")
        sbase = pl.multiple_of(wid * _SPW, 8)

        pltpu.sync_copy(x_hbm.at[pl.ds(sbase, _SPW)], idx_all)
        pltpu.sync_copy(pos_hbm, P_v)
        pltpu.sync_copy(pst_hbm, PST_v.at[pl.ds(0, _L)])
        pltpu.sync_copy(g_hbm, g_v)
        pltpu.sync_copy(b_hbm, b_v)

        iota16 = lax.iota(jnp.int32, 16)
        c64 = jnp.full((16,), _D, jnp.int32)
        c65 = jnp.full((16,), _D + 1, jnp.int32)
        c0 = jnp.zeros((16,), jnp.int32)
        c1 = jnp.ones((16,), jnp.int32)
        cmagic = jnp.full((16,), 0x5F3759DF, jnp.int32)

        gv = [g_v[pl.ds(t * 16, 16)] for t in range(8)]
        bv = [b_v[pl.ds(t * 16, 16)] for t in range(8)]

        bufs = (buf0, buf1, buf2)
        gsems = (gsem0, gsem1, gsem2)
        wsems = (wsem0, wsem1, wsem2)

        def start_gather(i, c):
            pltpu.async_copy(tab_hbm.at[idx_all.at[i, pl.ds(0, 128)]],
                             bufs[c].at[pl.ds(0, 128)], gsems[c])
            pltpu.async_copy(tab_hbm.at[idx_all.at[i, pl.ds(128, 72)]],
                             bufs[c].at[pl.ds(128, 72)], gsems[c])

        def wait_gather(c):
            pltpu.make_async_copy(
                tab_hbm.at[idx_all.at[0, pl.ds(0, 128)]],
                bufs[c].at[pl.ds(0, 128)], gsems[c]).wait()
            pltpu.make_async_copy(
                tab_hbm.at[idx_all.at[0, pl.ds(128, 72)]],
                bufs[c].at[pl.ds(128, 72)], gsems[c]).wait()

        def start_write(i, c):
            pltpu.async_copy(bufs[c].at[pl.ds(0, _L)],
                             out_hbm.at[sbase + i], wsems[c])

        def wait_write(c):
            pltpu.make_async_copy(bufs[c].at[pl.ds(0, _L)],
                                  out_hbm.at[0], wsems[c]).wait()

        def compute(c):
            r_ref = bufs[c]

            def do_group(r0, nrow):
                lvec = r0 + iota16
                S = plsc.load_gather(r_ref, [lvec, c64])
                Q = plsc.load_gather(r_ref, [lvec, c65])
                Sp = plsc.load_gather(PST_v, [lvec, c0])
                Qp = plsc.load_gather(PST_v, [lvec, c1])
                mu = (S + Sp) * (1.0 / _M)
                var = (Q + Qp) * (1.0 / _M) - mu * mu + 1e-5
                seed = cmagic - lax.shift_right_logical(
                    plsc.bitcast(var, jnp.int32), 1)
                y = plsc.bitcast(seed, jnp.float32)
                hx = 0.5 * var
                y = y * (1.5 - hx * y * y)
                y = y * (1.5 - hx * y * y)
                t1 = mu * y

                @plsc.parallel_loop(0, nrow, 1, unroll=4,
                                    carry=jnp.zeros((16,), jnp.int32))
                def rowp(j, jv):
                    jj = r0 + j
                    e = [r_ref[jj, pl.ds(t * 16, 16)] for t in range(4)]
                    p = [P_v[jj, pl.ds(t * 16, 16)] for t in range(4)]
                    ybc = y.at[jv].get(mode="promise_in_bounds")
                    tbc = t1.at[jv].get(mode="promise_in_bounds")
                    for t in range(4):
                        r_ref[jj, pl.ds(t * 16, 16)] = (
                            (e[t] * ybc - tbc) * gv[t] + bv[t])
                    for t in range(4):
                        r_ref[jj, pl.ds(_D + t * 16, 16)] = (
                            (p[t] * ybc - tbc) * gv[4 + t] + bv[4 + t])
                    return jv + 1

            def group(g, carry):
                do_group(g * 16, 16)
                return carry

            lax.fori_loop(0, _L // 16, group, 0)
            do_group(192, _L - 16 * (_L // 16))

        start_gather(0, 0)
        start_gather(1, 1)
        start_gather(2, 2)

        def step(ii, carry):
            for sub in range(3):
                i = ii * 3 + sub
                c = sub
                b = (sub + 2) % 3

                @pl.when(i < _SPW)
                def _():
                    wait_gather(c)
                    compute(c)
                    start_write(i, c)

                    @pl.when(jnp.logical_and(i >= 1, i + 2 < _SPW))
                    def _():
                        wait_write(b)
                        start_gather(i + 2, b)

            return carry

        lax.fori_loop(0, (_SPW + 2) // 3, step, 0)
        wait_write(0)
        wait_write(1)
        wait_write(2)

    return k(x, aug_tab, posT, pst, gamma, beta)


def kernel(x, table, pos_weight, ln_gamma, ln_beta):
    aug, posT, pst = _tc_prep(table, pos_weight)
    return _fused(x.astype(jnp.int32), aug, posT, pst, ln_gamma, ln_beta)
